# baseline (speedup 1.0000x reference)
import jax
import jax.numpy as jnp
from jax.experimental import pallas as pl
from jax.experimental.pallas import tpu as pltpu

_N = 4096
_LO, _HI = 1024, 3072
_BR = 512
_M0, _M1 = _LO // _BR, _HI // _BR


def _body(full_ref, left_ref, right_ref, bt_ref, o_ref):
    i = pl.program_id(0)
    in_rows = (i >= _M0) & (i < _M1)
    half = jnp.where((i % 2) == 1, _BR, 0)

    @pl.when(in_rows)
    def _mid():
        o_ref[:, :, : _LO] = left_ref[:, pl.ds(half, _BR), :]
        o_ref[:, :, _LO:_HI] = jnp.full((1, _BR, _HI - _LO), bt_ref[0, 0],
                                        jnp.float32)
        o_ref[:, :, _HI:] = right_ref[:, pl.ds(half, _BR), :]

    @pl.when(jnp.logical_not(in_rows))
    def _copy():
        o_ref[...] = full_ref[...]


def _full_idx(i):
    return (0, jnp.where((i >= _M0) & (i < _M1), _M0 - 1, i), 0)


def _slab_idx(col_block):
    def idx(i):
        return (0, jnp.clip(i, _M0, _M1 - 1) // 2, col_block)
    return idx


def kernel(B, Bt):
    return pl.pallas_call(
        _body,
        grid=(_N // _BR,),
        in_specs=[
            pl.BlockSpec((1, _BR, _N), _full_idx),
            pl.BlockSpec((1, 2 * _BR, _LO), _slab_idx(0)),
            pl.BlockSpec((1, 2 * _BR, _N - _HI), _slab_idx(_HI // (_N - _HI))),
            pl.BlockSpec(memory_space=pltpu.SMEM),
        ],
        out_specs=pl.BlockSpec((1, _BR, _N), lambda i: (0, i, 0)),
        out_shape=jax.ShapeDtypeStruct((1, _N, _N), jnp.float32),
    )(B, B, B, Bt)
